# baseline (speedup 1.0000x reference)
import functools
import jax
import jax.numpy as jnp
from jax import lax
from jax.experimental import pallas as pl
from jax.experimental.pallas import tpu as pltpu
from jax.experimental.pallas import tpu_sc as plsc

NC = 2
NS = 16
NW = NC * NS
LANES = 16
CHUNK = 128


def _sc_degree(dstb, ewb, n_pad, chunks):
    rpt = n_pad // NS

    def body(dst_hbm, ew_hbm, deg_out, dst_t, ew_t, zv, deg_sh):
        cid = lax.axis_index("c")
        sid = lax.axis_index("s")
        wid = cid * NS + sid

        def zb(i, _):
            zv[pl.ds(i * LANES, LANES)] = jnp.zeros((LANES,), jnp.float32)
            return 0
        lax.fori_loop(0, zv.shape[0] // LANES, zb, 0)
        pltpu.sync_copy(zv.at[pl.ds(0, rpt)], deg_sh.at[pl.ds(sid * rpt, rpt)])
        pltpu.sync_copy(dst_hbm.at[wid], dst_t)
        pltpu.sync_copy(ew_hbm.at[wid], ew_t)
        plsc.subcore_barrier()

        def acc(g, _):
            pltpu.sync_copy(ew_t.at[g], deg_sh.at[dst_t.at[g]], add=True)
            return 0
        lax.fori_loop(0, chunks, acc, 0)
        plsc.subcore_barrier()
        pltpu.sync_copy(deg_sh.at[pl.ds(sid * rpt, rpt)],
                        deg_out.at[cid].at[pl.ds(sid * rpt, rpt)])

    zlen = ((rpt + LANES - 1) // LANES) * LANES
    call = pl.kernel(
        body,
        out_type=jax.ShapeDtypeStruct((NC, n_pad), jnp.float32),
        mesh=plsc.VectorSubcoreMesh(core_axis_name="c", subcore_axis_name="s"),
        scratch_types=[
            pltpu.VMEM((chunks, CHUNK), jnp.int32),
            pltpu.VMEM((chunks, CHUNK), jnp.float32),
            pltpu.VMEM((zlen,), jnp.float32),
            pltpu.VMEM_SHARED((n_pad,), jnp.float32),
        ],
        compiler_params=pltpu.CompilerParams(use_tc_tiling_on_sc=False, needs_layout_passes=False),
    )
    return call(dstb, ewb)


def _sc_message(xw_i32, edata, dinv, n_pad, chunks):
    rpt = n_pad // NS
    npairs = chunks // 2
    W2 = 64

    def body(xw_hbm, ed_hbm, dinv_hbm, out_hbm,
             ed0, ed1, dst0, dst1, nrm0, nrm1, dinv_v, rb0, rb1, rows_f,
             stg0, stg1, gs0, gs1, scs, accum):
        cid = lax.axis_index("c")
        sid = lax.axis_index("s")
        wid = cid * NS + sid
        base = wid * chunks
        iota2 = lax.iota(jnp.int32, LANES) * 2

        def zrow(r, _):
            for j in range(8):
                rows_f[r, pl.ds(j * LANES, LANES)] = jnp.zeros((LANES,), jnp.float32)
            return 0
        lax.fori_loop(0, CHUNK, zrow, 0)
        for t in range(rpt // CHUNK):
            pltpu.sync_copy(rows_f, accum.at[pl.ds(sid * rpt + t * CHUNK, CHUNK)])
        pltpu.sync_copy(dinv_hbm, dinv_v)
        plsc.subcore_barrier()

        pltpu.sync_copy(ed_hbm.at[base + 0], ed0)
        pltpu.sync_copy(ed_hbm.at[base + 1], ed1)
        pltpu.async_copy(xw_hbm.at[ed0.at[pl.ds(0, CHUNK)]], rb0, gs0)

        def pair(i, _):
            g0 = 2 * i
            @pl.when(i > 0)
            def _():
                pltpu.make_async_copy(ed_hbm.at[base], ed1, stg1).wait()
            gather1 = pltpu.async_copy(xw_hbm.at[ed1.at[pl.ds(0, CHUNK)]], rb1, gs1)

            def gwait0():
                pltpu.make_async_copy(xw_hbm.at[ed0.at[pl.ds(0, CHUNK)]], rb0, gs0).wait()
            gwait0()
            for j in range(8):
                sl = pl.ds(j * LANES, LANES)
                s16 = ed0[pl.ds(j * LANES, LANES)]
                d16 = ed0[pl.ds(CHUNK + j * LANES, LANES)]
                w16 = plsc.bitcast(ed0[pl.ds(2 * CHUNK + j * LANES, LANES)], jnp.float32)
                dst0[sl] = d16
                nrm0[sl] = plsc.load_gather(dinv_v, [s16]) * w16 * \
                    plsc.load_gather(dinv_v, [d16])
            @pl.when(i > 0)
            def _():
                pltpu.make_async_copy(rows_f, accum.at[dst0], scs).wait()

            def scale0(k, _):
                k16 = jnp.full((LANES,), k, jnp.int32)
                nv = plsc.load_gather(nrm0, [k16])
                for j in range(4):
                    v = rb0[k, pl.ds(j * LANES, LANES)]
                    fe = plsc.bitcast(v << 16, jnp.float32) * nv
                    fo = plsc.bitcast(v & jnp.int32(-65536), jnp.float32) * nv
                    cbase = 32 * j + iota2
                    plsc.store_scatter(rows_f, [k16, cbase], fe)
                    plsc.store_scatter(rows_f, [k16, cbase + 1], fo)
                return 0
            lax.fori_loop(0, CHUNK, scale0, 0)
            pltpu.async_copy(rows_f, accum.at[dst0], scs, add=True)
            pltpu.async_copy(ed_hbm.at[base + g0 + 2], ed0, stg0)

            pltpu.make_async_copy(ed_hbm.at[base], ed0, stg0).wait()
            pltpu.async_copy(xw_hbm.at[ed0.at[pl.ds(0, CHUNK)]], rb0, gs0)
            gather1.wait()
            for j in range(8):
                sl = pl.ds(j * LANES, LANES)
                s16 = ed1[pl.ds(j * LANES, LANES)]
                d16 = ed1[pl.ds(CHUNK + j * LANES, LANES)]
                w16 = plsc.bitcast(ed1[pl.ds(2 * CHUNK + j * LANES, LANES)], jnp.float32)
                dst1[sl] = d16
                nrm1[sl] = plsc.load_gather(dinv_v, [s16]) * w16 * \
                    plsc.load_gather(dinv_v, [d16])
            pltpu.make_async_copy(rows_f, accum.at[dst1], scs).wait()

            def scale1(k, _):
                k16 = jnp.full((LANES,), k, jnp.int32)
                nv = plsc.load_gather(nrm1, [k16])
                for j in range(4):
                    v = rb1[k, pl.ds(j * LANES, LANES)]
                    fe = plsc.bitcast(v << 16, jnp.float32) * nv
                    fo = plsc.bitcast(v & jnp.int32(-65536), jnp.float32) * nv
                    cbase = 32 * j + iota2
                    plsc.store_scatter(rows_f, [k16, cbase], fe)
                    plsc.store_scatter(rows_f, [k16, cbase + 1], fo)
                return 0
            lax.fori_loop(0, CHUNK, scale1, 0)
            pltpu.async_copy(rows_f, accum.at[dst1], scs, add=True)
            pltpu.async_copy(ed_hbm.at[base + g0 + 3], ed1, stg1)
            return 0
        lax.fori_loop(0, npairs, pair, 0)

        pltpu.make_async_copy(xw_hbm.at[ed0.at[pl.ds(0, CHUNK)]], rb0, gs0).wait()
        pltpu.make_async_copy(ed_hbm.at[base], ed1, stg1).wait()
        pltpu.make_async_copy(rows_f, accum.at[dst1], scs).wait()
        plsc.subcore_barrier()

        pltpu.sync_copy(accum.at[pl.ds(sid * rpt, rpt)],
                        out_hbm.at[cid].at[pl.ds(sid * rpt, rpt)])

    call = pl.kernel(
        body,
        out_type=jax.ShapeDtypeStruct((NC, n_pad, 128), jnp.float32),
        mesh=plsc.VectorSubcoreMesh(core_axis_name="c", subcore_axis_name="s"),
        scratch_types=[
            pltpu.VMEM((3 * CHUNK,), jnp.int32),
            pltpu.VMEM((3 * CHUNK,), jnp.int32),
            pltpu.VMEM((CHUNK,), jnp.int32),
            pltpu.VMEM((CHUNK,), jnp.int32),
            pltpu.VMEM((CHUNK,), jnp.float32),
            pltpu.VMEM((CHUNK,), jnp.float32),
            pltpu.VMEM((n_pad,), jnp.float32),
            pltpu.VMEM((CHUNK, W2), jnp.int32),
            pltpu.VMEM((CHUNK, W2), jnp.int32),
            pltpu.VMEM((CHUNK, 128), jnp.float32),
            pltpu.SemaphoreType.DMA,
            pltpu.SemaphoreType.DMA,
            pltpu.SemaphoreType.DMA,
            pltpu.SemaphoreType.DMA,
            pltpu.SemaphoreType.DMA,
            pltpu.VMEM_SHARED((n_pad, 128), jnp.float32),
        ],
        compiler_params=pltpu.CompilerParams(use_tc_tiling_on_sc=False, needs_layout_passes=False),
    )
    return call(xw_i32, edata, dinv)


def _tc_prep(degp, x_p, W0, n_pad):
    nb = n_pad // 128

    def body(degp_ref, x_ref, w_ref, xw_ref, dinv_ref):
        deg = degp_ref[0, :] + degp_ref[1, :]
        dinv_ref[0, 0, :] = lax.rsqrt(deg)
        xw_ref[...] = jnp.dot(x_ref[...], w_ref[...],
                              preferred_element_type=jnp.float32).astype(jnp.bfloat16)

    return pl.pallas_call(
        body,
        grid=(nb,),
        in_specs=[
            pl.BlockSpec((NC, 128), lambda b: (0, b)),
            pl.BlockSpec((128, 128), lambda b: (b, 0)),
            pl.BlockSpec((128, 128), lambda b: (0, 0)),
        ],
        out_specs=[
            pl.BlockSpec((128, 128), lambda b: (b, 0)),
            pl.BlockSpec((1, 1, 128), lambda b: (b, 0, 0)),
        ],
        out_shape=[
            jax.ShapeDtypeStruct((n_pad, 128), jnp.bfloat16),
            jax.ShapeDtypeStruct((nb, 1, 128), jnp.float32),
        ],
    )(degp, x_p, W0)


def _tc_combine(parts, b, W, n_pad):
    nb = n_pad // 128

    def body(p_ref, b_ref, w_ref, xw_ref):
        h = jnp.maximum(p_ref[0] + p_ref[1] + b_ref[...], 0.0)
        xw_ref[...] = jnp.dot(h, w_ref[...],
                              preferred_element_type=jnp.float32).astype(jnp.bfloat16)

    return pl.pallas_call(
        body,
        grid=(nb,),
        in_specs=[
            pl.BlockSpec((NC, 128, 128), lambda i: (0, i, 0)),
            pl.BlockSpec((1, 128), lambda i: (0, 0)),
            pl.BlockSpec((128, 128), lambda i: (0, 0)),
        ],
        out_specs=pl.BlockSpec((128, 128), lambda i: (i, 0)),
        out_shape=jax.ShapeDtypeStruct((n_pad, 128), jnp.bfloat16),
    )(parts, b, W)


def _tc_final(parts, b, n_pad):
    nb = n_pad // 128

    def body(p_ref, b_ref, o_ref):
        o_ref[...] = jnp.maximum(p_ref[0] + p_ref[1] + b_ref[...], 0.0)

    return pl.pallas_call(
        body,
        grid=(nb,),
        in_specs=[
            pl.BlockSpec((NC, 128, 128), lambda i: (0, i, 0)),
            pl.BlockSpec((1, 128), lambda i: (0, 0)),
        ],
        out_specs=pl.BlockSpec((128, 128), lambda i: (i, 0)),
        out_shape=jax.ShapeDtypeStruct((n_pad, 128), jnp.float32),
    )(parts, b)


def kernel(x, edge_index, edge_weight, W0, b0, W1, b1):
    N, D = x.shape
    E = edge_weight.shape[0]
    n_pad = ((N + NS * 128 - 1) // (NS * 128)) * (NS * 128)

    e_tot = E + n_pad
    chunks = (e_tot + NW * CHUNK - 1) // (NW * CHUNK)
    chunks = chunks + (chunks % 2)
    e_pad = NW * chunks * CHUNK
    loop_idx = jnp.arange(n_pad, dtype=jnp.int32)
    zpad = jnp.zeros((e_pad - e_tot,), jnp.int32)
    src = jnp.concatenate([edge_index[0], loop_idx, zpad]).reshape(NW, chunks, CHUNK)
    dst = jnp.concatenate([edge_index[1], loop_idx, zpad]).reshape(NW, chunks, CHUNK)
    ew = jnp.concatenate([
        edge_weight, jnp.ones((n_pad,), jnp.float32),
        jnp.zeros((e_pad - e_tot,), jnp.float32),
    ]).reshape(NW, chunks, CHUNK)
    edata = jnp.concatenate(
        [src.reshape(-1, CHUNK), dst.reshape(-1, CHUNK),
         ew.reshape(-1, CHUNK).view(jnp.int32)], axis=1)
    edata = jnp.pad(edata, ((0, 2), (0, 0)))

    x_p = jnp.pad(x, ((0, n_pad - N), (0, 0)))

    degp = _sc_degree(dst, ew, n_pad, chunks)
    xw0, dinv2d = _tc_prep(degp, x_p, W0, n_pad)
    dinv = dinv2d.reshape(n_pad)

    p0 = _sc_message(xw0.view(jnp.int32), edata, dinv, n_pad, chunks)
    xw1 = _tc_combine(p0, b0.reshape(1, 128), W1, n_pad)

    p1 = _sc_message(xw1.view(jnp.int32), edata, dinv, n_pad, chunks)
    out = _tc_final(p1, b1.reshape(1, 128), n_pad)
    return out[:N]


# speedup vs baseline: 1.0004x; 1.0004x over previous
import functools
import jax
import jax.numpy as jnp
from jax import lax
from jax.experimental import pallas as pl
from jax.experimental.pallas import tpu as pltpu
from jax.experimental.pallas import tpu_sc as plsc

NC = 2
NS = 16
NW = NC * NS
LANES = 16
CHUNK = 128


def _sc_degree(dstb, ewb, n_pad, chunks):
    rpt = n_pad // NS

    def body(dst_hbm, ew_hbm, deg_out, dst_t, ew_t, zv, deg_sh):
        cid = lax.axis_index("c")
        sid = lax.axis_index("s")
        wid = cid * NS + sid

        def zb(i, _):
            zv[pl.ds(i * LANES, LANES)] = jnp.zeros((LANES,), jnp.float32)
            return 0
        lax.fori_loop(0, zv.shape[0] // LANES, zb, 0)
        pltpu.sync_copy(zv.at[pl.ds(0, rpt)], deg_sh.at[pl.ds(sid * rpt, rpt)])
        pltpu.sync_copy(dst_hbm.at[wid], dst_t)
        pltpu.sync_copy(ew_hbm.at[wid], ew_t)
        plsc.subcore_barrier()

        def acc(g, _):
            pltpu.sync_copy(ew_t.at[g], deg_sh.at[dst_t.at[g]], add=True)
            return 0
        lax.fori_loop(0, chunks, acc, 0)
        plsc.subcore_barrier()
        pltpu.sync_copy(deg_sh.at[pl.ds(sid * rpt, rpt)],
                        deg_out.at[cid].at[pl.ds(sid * rpt, rpt)])

    zlen = ((rpt + LANES - 1) // LANES) * LANES
    call = pl.kernel(
        body,
        out_type=jax.ShapeDtypeStruct((NC, n_pad), jnp.float32),
        mesh=plsc.VectorSubcoreMesh(core_axis_name="c", subcore_axis_name="s"),
        scratch_types=[
            pltpu.VMEM((chunks, CHUNK), jnp.int32),
            pltpu.VMEM((chunks, CHUNK), jnp.float32),
            pltpu.VMEM((zlen,), jnp.float32),
            pltpu.VMEM_SHARED((n_pad,), jnp.float32),
        ],
        compiler_params=pltpu.CompilerParams(use_tc_tiling_on_sc=False, needs_layout_passes=False),
    )
    return call(dstb, ewb)


def _sc_message(xw_i32, edata, dinv, n_pad, chunks):
    rpt = n_pad // NS
    npairs = chunks // 2
    W2 = 64

    def body(xw_hbm, ed_hbm, dinv_hbm, out_hbm,
             ed0, ed1, dst0, dst1, nrm0, nrm1, dinv_v, rb0, rb1, rows_f,
             stg0, stg1, gs0, gs1, scs, accum):
        cid = lax.axis_index("c")
        sid = lax.axis_index("s")
        wid = cid * NS + sid
        base = wid * chunks
        iota2 = lax.iota(jnp.int32, LANES) * 2

        def zrow(r, _):
            for j in range(8):
                rows_f[r, pl.ds(j * LANES, LANES)] = jnp.zeros((LANES,), jnp.float32)
            return 0
        lax.fori_loop(0, CHUNK, zrow, 0)
        for t in range(rpt // CHUNK):
            pltpu.sync_copy(rows_f, accum.at[pl.ds(sid * rpt + t * CHUNK, CHUNK)])
        pltpu.sync_copy(dinv_hbm, dinv_v)
        plsc.subcore_barrier()

        pltpu.sync_copy(ed_hbm.at[base + 0], ed0)
        pltpu.sync_copy(ed_hbm.at[base + 1], ed1)
        pltpu.async_copy(xw_hbm.at[ed0.at[pl.ds(0, CHUNK // 2)]], rb0.at[pl.ds(0, CHUNK // 2)], gs0)
        pltpu.async_copy(xw_hbm.at[ed0.at[pl.ds(CHUNK // 2, CHUNK // 2)]], rb0.at[pl.ds(CHUNK // 2, CHUNK // 2)], gs0)

        def pair(i, _):
            g0 = 2 * i
            @pl.when(i > 0)
            def _():
                pltpu.make_async_copy(ed_hbm.at[base], ed1, stg1).wait()
            pltpu.async_copy(xw_hbm.at[ed1.at[pl.ds(0, CHUNK // 2)]], rb1.at[pl.ds(0, CHUNK // 2)], gs1)
            pltpu.async_copy(xw_hbm.at[ed1.at[pl.ds(CHUNK // 2, CHUNK // 2)]], rb1.at[pl.ds(CHUNK // 2, CHUNK // 2)], gs1)

            def gwait0():
                pltpu.make_async_copy(xw_hbm.at[ed0.at[pl.ds(0, CHUNK)]], rb0, gs0).wait()
            gwait0()
            for j in range(8):
                sl = pl.ds(j * LANES, LANES)
                s16 = ed0[pl.ds(j * LANES, LANES)]
                d16 = ed0[pl.ds(CHUNK + j * LANES, LANES)]
                w16 = plsc.bitcast(ed0[pl.ds(2 * CHUNK + j * LANES, LANES)], jnp.float32)
                dst0[sl] = d16
                nrm0[sl] = plsc.load_gather(dinv_v, [s16]) * w16 * \
                    plsc.load_gather(dinv_v, [d16])
            @pl.when(i > 0)
            def _():
                pltpu.make_async_copy(rows_f, accum.at[dst0], scs).wait()

            def scale0(k, _):
                k16 = jnp.full((LANES,), k, jnp.int32)
                nv = plsc.load_gather(nrm0, [k16])
                for j in range(4):
                    v = rb0[k, pl.ds(j * LANES, LANES)]
                    fe = plsc.bitcast(v << 16, jnp.float32) * nv
                    fo = plsc.bitcast(v & jnp.int32(-65536), jnp.float32) * nv
                    cbase = 32 * j + iota2
                    plsc.store_scatter(rows_f, [k16, cbase], fe)
                    plsc.store_scatter(rows_f, [k16, cbase + 1], fo)
                return 0
            lax.fori_loop(0, CHUNK, scale0, 0)
            pltpu.async_copy(rows_f, accum.at[dst0], scs, add=True)
            pltpu.async_copy(ed_hbm.at[base + g0 + 2], ed0, stg0)

            pltpu.make_async_copy(ed_hbm.at[base], ed0, stg0).wait()
            pltpu.async_copy(xw_hbm.at[ed0.at[pl.ds(0, CHUNK // 2)]], rb0.at[pl.ds(0, CHUNK // 2)], gs0)
            pltpu.async_copy(xw_hbm.at[ed0.at[pl.ds(CHUNK // 2, CHUNK // 2)]], rb0.at[pl.ds(CHUNK // 2, CHUNK // 2)], gs0)
            pltpu.make_async_copy(xw_hbm.at[ed1.at[pl.ds(0, CHUNK)]], rb1, gs1).wait()
            for j in range(8):
                sl = pl.ds(j * LANES, LANES)
                s16 = ed1[pl.ds(j * LANES, LANES)]
                d16 = ed1[pl.ds(CHUNK + j * LANES, LANES)]
                w16 = plsc.bitcast(ed1[pl.ds(2 * CHUNK + j * LANES, LANES)], jnp.float32)
                dst1[sl] = d16
                nrm1[sl] = plsc.load_gather(dinv_v, [s16]) * w16 * \
                    plsc.load_gather(dinv_v, [d16])
            pltpu.make_async_copy(rows_f, accum.at[dst1], scs).wait()

            def scale1(k, _):
                k16 = jnp.full((LANES,), k, jnp.int32)
                nv = plsc.load_gather(nrm1, [k16])
                for j in range(4):
                    v = rb1[k, pl.ds(j * LANES, LANES)]
                    fe = plsc.bitcast(v << 16, jnp.float32) * nv
                    fo = plsc.bitcast(v & jnp.int32(-65536), jnp.float32) * nv
                    cbase = 32 * j + iota2
                    plsc.store_scatter(rows_f, [k16, cbase], fe)
                    plsc.store_scatter(rows_f, [k16, cbase + 1], fo)
                return 0
            lax.fori_loop(0, CHUNK, scale1, 0)
            pltpu.async_copy(rows_f, accum.at[dst1], scs, add=True)
            pltpu.async_copy(ed_hbm.at[base + g0 + 3], ed1, stg1)
            return 0
        lax.fori_loop(0, npairs, pair, 0)

        pltpu.make_async_copy(xw_hbm.at[ed0.at[pl.ds(0, CHUNK)]], rb0, gs0).wait()
        pltpu.make_async_copy(ed_hbm.at[base], ed1, stg1).wait()
        pltpu.make_async_copy(rows_f, accum.at[dst1], scs).wait()
        plsc.subcore_barrier()

        pltpu.sync_copy(accum.at[pl.ds(sid * rpt, rpt)],
                        out_hbm.at[cid].at[pl.ds(sid * rpt, rpt)])

    call = pl.kernel(
        body,
        out_type=jax.ShapeDtypeStruct((NC, n_pad, 128), jnp.float32),
        mesh=plsc.VectorSubcoreMesh(core_axis_name="c", subcore_axis_name="s"),
        scratch_types=[
            pltpu.VMEM((3 * CHUNK,), jnp.int32),
            pltpu.VMEM((3 * CHUNK,), jnp.int32),
            pltpu.VMEM((CHUNK,), jnp.int32),
            pltpu.VMEM((CHUNK,), jnp.int32),
            pltpu.VMEM((CHUNK,), jnp.float32),
            pltpu.VMEM((CHUNK,), jnp.float32),
            pltpu.VMEM((n_pad,), jnp.float32),
            pltpu.VMEM((CHUNK, W2), jnp.int32),
            pltpu.VMEM((CHUNK, W2), jnp.int32),
            pltpu.VMEM((CHUNK, 128), jnp.float32),
            pltpu.SemaphoreType.DMA,
            pltpu.SemaphoreType.DMA,
            pltpu.SemaphoreType.DMA,
            pltpu.SemaphoreType.DMA,
            pltpu.SemaphoreType.DMA,
            pltpu.VMEM_SHARED((n_pad, 128), jnp.float32),
        ],
        compiler_params=pltpu.CompilerParams(use_tc_tiling_on_sc=False, needs_layout_passes=False),
    )
    return call(xw_i32, edata, dinv)


def _tc_prep(degp, x_p, W0, n_pad):
    nb = n_pad // 128

    def body(degp_ref, x_ref, w_ref, xw_ref, dinv_ref):
        deg = degp_ref[0, :] + degp_ref[1, :]
        dinv_ref[0, 0, :] = lax.rsqrt(deg)
        xw_ref[...] = jnp.dot(x_ref[...], w_ref[...],
                              preferred_element_type=jnp.float32).astype(jnp.bfloat16)

    return pl.pallas_call(
        body,
        grid=(nb,),
        in_specs=[
            pl.BlockSpec((NC, 128), lambda b: (0, b)),
            pl.BlockSpec((128, 128), lambda b: (b, 0)),
            pl.BlockSpec((128, 128), lambda b: (0, 0)),
        ],
        out_specs=[
            pl.BlockSpec((128, 128), lambda b: (b, 0)),
            pl.BlockSpec((1, 1, 128), lambda b: (b, 0, 0)),
        ],
        out_shape=[
            jax.ShapeDtypeStruct((n_pad, 128), jnp.bfloat16),
            jax.ShapeDtypeStruct((nb, 1, 128), jnp.float32),
        ],
    )(degp, x_p, W0)


def _tc_combine(parts, b, W, n_pad):
    nb = n_pad // 128

    def body(p_ref, b_ref, w_ref, xw_ref):
        h = jnp.maximum(p_ref[0] + p_ref[1] + b_ref[...], 0.0)
        xw_ref[...] = jnp.dot(h, w_ref[...],
                              preferred_element_type=jnp.float32).astype(jnp.bfloat16)

    return pl.pallas_call(
        body,
        grid=(nb,),
        in_specs=[
            pl.BlockSpec((NC, 128, 128), lambda i: (0, i, 0)),
            pl.BlockSpec((1, 128), lambda i: (0, 0)),
            pl.BlockSpec((128, 128), lambda i: (0, 0)),
        ],
        out_specs=pl.BlockSpec((128, 128), lambda i: (i, 0)),
        out_shape=jax.ShapeDtypeStruct((n_pad, 128), jnp.bfloat16),
    )(parts, b, W)


def _tc_final(parts, b, n_pad):
    nb = n_pad // 128

    def body(p_ref, b_ref, o_ref):
        o_ref[...] = jnp.maximum(p_ref[0] + p_ref[1] + b_ref[...], 0.0)

    return pl.pallas_call(
        body,
        grid=(nb,),
        in_specs=[
            pl.BlockSpec((NC, 128, 128), lambda i: (0, i, 0)),
            pl.BlockSpec((1, 128), lambda i: (0, 0)),
        ],
        out_specs=pl.BlockSpec((128, 128), lambda i: (i, 0)),
        out_shape=jax.ShapeDtypeStruct((n_pad, 128), jnp.float32),
    )(parts, b)


def kernel(x, edge_index, edge_weight, W0, b0, W1, b1):
    N, D = x.shape
    E = edge_weight.shape[0]
    n_pad = ((N + NS * 128 - 1) // (NS * 128)) * (NS * 128)

    e_tot = E + n_pad
    chunks = (e_tot + NW * CHUNK - 1) // (NW * CHUNK)
    chunks = chunks + (chunks % 2)
    e_pad = NW * chunks * CHUNK
    loop_idx = jnp.arange(n_pad, dtype=jnp.int32)
    zpad = jnp.zeros((e_pad - e_tot,), jnp.int32)
    src = jnp.concatenate([edge_index[0], loop_idx, zpad]).reshape(NW, chunks, CHUNK)
    dst = jnp.concatenate([edge_index[1], loop_idx, zpad]).reshape(NW, chunks, CHUNK)
    ew = jnp.concatenate([
        edge_weight, jnp.ones((n_pad,), jnp.float32),
        jnp.zeros((e_pad - e_tot,), jnp.float32),
    ]).reshape(NW, chunks, CHUNK)
    edata = jnp.concatenate(
        [src.reshape(-1, CHUNK), dst.reshape(-1, CHUNK),
         ew.reshape(-1, CHUNK).view(jnp.int32)], axis=1)
    edata = jnp.pad(edata, ((0, 2), (0, 0)))

    x_p = jnp.pad(x, ((0, n_pad - N), (0, 0)))

    degp = _sc_degree(dst, ew, n_pad, chunks)
    xw0, dinv2d = _tc_prep(degp, x_p, W0, n_pad)
    dinv = dinv2d.reshape(n_pad)

    p0 = _sc_message(xw0.view(jnp.int32), edata, dinv, n_pad, chunks)
    xw1 = _tc_combine(p0, b0.reshape(1, 128), W1, n_pad)

    p1 = _sc_message(xw1.view(jnp.int32), edata, dinv, n_pad, chunks)
    out = _tc_final(p1, b1.reshape(1, 128), n_pad)
    return out[:N]
